# baseline (speedup 1.0000x reference)
import functools

import jax
import jax.numpy as jnp
from jax import lax
from jax.experimental import pallas as pl
from jax.experimental.pallas import tpu as pltpu
from jax.experimental.pallas import tpu_sc as plsc

_NUM_FIELDS = 26
_FIELD_DIM = 100000
_EMBED_DIM = 16

_info = plsc.get_sparse_core_info()
_NC, _NS = _info.num_cores, _info.num_subcores
_NW = _NC * _NS


def _make_gather(num_rows: int, chunk: int):
  assert num_rows % _NW == 0
  per_w = num_rows // _NW
  assert per_w % chunk == 0 and chunk % 8 == 0
  n_chunks = per_w // chunk
  mesh = plsc.VectorSubcoreMesh(core_axis_name="c", subcore_axis_name="s")

  @functools.partial(
      pl.kernel,
      mesh=mesh,
      out_type=jax.ShapeDtypeStruct((num_rows, _EMBED_DIM), jnp.float32),
      scratch_types=[
          pltpu.VMEM((chunk,), jnp.int32),
          pltpu.VMEM((chunk, _EMBED_DIM), jnp.float32),
          pltpu.SemaphoreType.DMA,
      ],
      compiler_params=pltpu.CompilerParams(use_tc_tiling_on_sc=False),
  )
  def k(idx_hbm, table_hbm, out_hbm, idx_v, rows_v, sem):
    wid = lax.axis_index("s") * _NC + lax.axis_index("c")
    base = wid * per_w

    def body(i, _):
      off = base + i * chunk
      pltpu.sync_copy(idx_hbm.at[pl.ds(off, chunk)], idx_v)
      pltpu.async_copy(table_hbm.at[idx_v], rows_v, sem).wait()
      pltpu.sync_copy(rows_v, out_hbm.at[pl.ds(off, chunk)])
      return ()

    lax.fori_loop(0, n_chunks, body, (), unroll=False)

  return k


def kernel(x, weight):
  b, f = x.shape
  offsets = jnp.arange(f, dtype=jnp.int32) * _FIELD_DIM
  idx = (x + offsets[None, :]).reshape(-1)
  gather = _make_gather(b * f, 3328)
  out = gather(idx, weight)
  return out.reshape(b, f, _EMBED_DIM)


# speedup vs baseline: 1.0016x; 1.0016x over previous
import functools

import jax
import jax.numpy as jnp
from jax import lax
from jax.experimental import pallas as pl
from jax.experimental.pallas import tpu as pltpu
from jax.experimental.pallas import tpu_sc as plsc

_NUM_FIELDS = 26
_FIELD_DIM = 100000
_EMBED_DIM = 16

_info = plsc.get_sparse_core_info()
_NC, _NS = _info.num_cores, _info.num_subcores
_NW = _NC * _NS


def _make_gather(num_rows: int, chunk: int):
  assert num_rows % _NW == 0
  per_w = num_rows // _NW
  assert per_w % chunk == 0 and chunk % 8 == 0
  n_chunks = per_w // chunk
  mesh = plsc.VectorSubcoreMesh(core_axis_name="c", subcore_axis_name="s")

  @functools.partial(
      pl.kernel,
      mesh=mesh,
      out_type=jax.ShapeDtypeStruct((num_rows, _EMBED_DIM), jnp.float32),
      scratch_types=[
          pltpu.VMEM((per_w,), jnp.int32),
          pltpu.VMEM((chunk, _EMBED_DIM), jnp.float32),
          pltpu.VMEM((chunk, _EMBED_DIM), jnp.float32),
          pltpu.SemaphoreType.DMA,
          pltpu.SemaphoreType.DMA,
          pltpu.SemaphoreType.DMA,
          pltpu.SemaphoreType.DMA,
      ],
      compiler_params=pltpu.CompilerParams(use_tc_tiling_on_sc=False),
  )
  def k(idx_hbm, table_hbm, out_hbm, idx_v, rows0, rows1, g0, g1, s0, s1):
    wid = lax.axis_index("s") * _NC + lax.axis_index("c")
    base = wid * per_w
    pltpu.sync_copy(idx_hbm.at[pl.ds(base, per_w)], idx_v)

    rows = (rows0, rows1)
    gsem = (g0, g1)
    ssem = (s0, s1)
    gathers = [None] * n_chunks
    scatters = [None] * n_chunks
    for i in range(n_chunks):
      b = i % 2
      if i >= 2:
        scatters[i - 2].wait()
      gathers[i] = pltpu.async_copy(
          table_hbm.at[idx_v.at[pl.ds(i * chunk, chunk)]], rows[b], gsem[b])
      if i >= 1:
        gathers[i - 1].wait()
        scatters[i - 1] = pltpu.async_copy(
            rows[1 - b], out_hbm.at[pl.ds(base + (i - 1) * chunk, chunk)],
            ssem[1 - b])
    last = n_chunks - 1
    gathers[last].wait()
    scatters[last] = pltpu.async_copy(
        rows[last % 2], out_hbm.at[pl.ds(base + last * chunk, chunk)],
        ssem[last % 2])
    if n_chunks >= 2:
      scatters[last - 1].wait()
    scatters[last].wait()

  return k


def kernel(x, weight):
  b, f = x.shape
  offsets = jnp.arange(f, dtype=jnp.int32) * _FIELD_DIM
  idx = (x + offsets[None, :]).reshape(-1)
  gather = _make_gather(b * f, 3328)
  out = gather(idx, weight)
  return out.reshape(b, f, _EMBED_DIM)


# speedup vs baseline: 1.2318x; 1.2299x over previous
import functools

import jax
import jax.numpy as jnp
from jax import lax
from jax.experimental import pallas as pl
from jax.experimental.pallas import tpu as pltpu
from jax.experimental.pallas import tpu_sc as plsc

_NUM_FIELDS = 26
_FIELD_DIM = 100000
_EMBED_DIM = 16

_info = plsc.get_sparse_core_info()
_NC, _NS = _info.num_cores, _info.num_subcores
_NW = _NC * _NS


def _make_lookup(batch: int):
  assert batch % _NW == 0
  bpw = batch // _NW
  mesh = plsc.VectorSubcoreMesh(core_axis_name="c", subcore_axis_name="s")
  d = _EMBED_DIM

  @functools.partial(
      pl.kernel,
      mesh=mesh,
      out_type=jax.ShapeDtypeStruct((_NUM_FIELDS, d, batch), jnp.float32),
      scratch_types=[
          pltpu.VMEM((bpw,), jnp.int32),
          pltpu.VMEM((bpw, d), jnp.float32),
          pltpu.VMEM((d, bpw), jnp.float32),
          pltpu.SemaphoreType.DMA,
      ],
      compiler_params=pltpu.CompilerParams(use_tc_tiling_on_sc=False, needs_layout_passes=False),
  )
  def k(idx_hbm, table_hbm, out_hbm, idx_v, rows_v, cols_v, gsem):
    wid = lax.axis_index("s") * _NC + lax.axis_index("c")
    b0 = wid * bpw
    lane = jax.lax.broadcasted_iota(jnp.int32, (16,), 0)

    def body(f, _):
      pltpu.sync_copy(idx_hbm.at[f, pl.ds(b0, bpw)], idx_v)
      pltpu.async_copy(table_hbm.at[idx_v], rows_v, gsem).wait()
      def col(r16, _):
        rows = r16 * 16 + lane
        for c in range(d):
          cols_v[c, pl.ds(r16 * 16, 16)] = plsc.load_gather(
              rows_v, [rows, jnp.full((16,), c, jnp.int32)])
        return ()

      lax.fori_loop(0, bpw // 16, col, (), unroll=False)
      pltpu.sync_copy(cols_v, out_hbm.at[f, :, pl.ds(b0, bpw)])
      return ()

    lax.fori_loop(0, _NUM_FIELDS, body, (), unroll=False)

  return k


def kernel(x, weight):
  b, f = x.shape
  offsets = jnp.arange(f, dtype=jnp.int32) * _FIELD_DIM
  idx_t = x.T + offsets[:, None]
  lookup = _make_lookup(b)
  out_t = lookup(idx_t, weight)
  return out_t.transpose(2, 0, 1)
